# baseline (speedup 1.0000x reference)
import jax
import jax.numpy as jnp
from jax.experimental import pallas as pl
from jax.experimental.pallas import tpu as pltpu


def _mm(a3, w):
    B, T, K = a3.shape
    return jnp.dot(a3.reshape(B * T, K), w,
                   preferred_element_type=jnp.float32).reshape(B, T, w.shape[1])


def _cnn_rnn_body(emb_ref, feat_ref,
                  w1_ref, b1_ref,
                  w2_ref, b2_ref,
                  wih1_ref, whh1_ref, bg1_ref,
                  wihh2_ref, bg2_ref,
                  wfco_ref, wfcf_ref, bfc_ref,
                  out_ref):
    emb = emb_ref[...]
    Bb, L, E = emb.shape
    T = L - 2
    C1 = w1_ref.shape[1]
    H = whh1_ref.shape[0]

    win1 = jnp.concatenate(
        [emb[:, 0:T, :], emb[:, 1:T + 1, :], emb[:, 2:T + 2, :]], axis=-1)
    c1 = jax.nn.sigmoid(_mm(win1, w1_ref[...]) + b1_ref[...])

    zpad = jnp.zeros((Bb, 1, C1), jnp.float32)
    c1p = jnp.concatenate([zpad, c1, zpad], axis=1)
    win2 = jnp.concatenate(
        [c1p[:, 0:T, :], c1p[:, 1:T + 1, :], c1p[:, 2:T + 2, :]], axis=-1)
    c2 = jax.nn.sigmoid(_mm(win2, w2_ref[...]) + b2_ref[...])

    gates_x1 = _mm(c2, wih1_ref[...]) + bg1_ref[...]
    whh1 = whh1_ref[...]
    wih2 = wihh2_ref[0:H, :]
    whh2 = wihh2_ref[H:2 * H, :]
    bg2 = bg2_ref[...]

    def apply_gates(gates, c_prev):
        sg = jax.nn.sigmoid(gates)
        tg = jnp.tanh(gates)
        i = sg[:, 0:H]
        f = sg[:, H:2 * H]
        o = sg[:, 3 * H:4 * H]
        g = tg[:, 2 * H:3 * H]
        c_new = f * c_prev + i * g
        h_new = o * jnp.tanh(c_new)
        return h_new, c_new

    h1 = jnp.zeros((Bb, H), jnp.float32)
    c1s = jnp.zeros((Bb, H), jnp.float32)
    hs1 = []
    for t in range(T):
        gates1 = gates_x1[:, t, :] + jnp.dot(
            h1, whh1, preferred_element_type=jnp.float32)
        h1, c1s = apply_gates(gates1, c1s)
        hs1.append(h1)

    h1all = jnp.concatenate(hs1, axis=-1).reshape(Bb, T, H)
    gates_x2 = _mm(h1all, wih2) + bg2

    h2 = jnp.zeros((Bb, H), jnp.float32)
    c2s = jnp.zeros((Bb, H), jnp.float32)
    hs2 = []
    for t in range(T):
        gates2 = gates_x2[:, t, :] + jnp.dot(
            h2, whh2, preferred_element_type=jnp.float32)
        h2, c2s = apply_gates(gates2, c2s)
        hs2.append(h2)

    hflat = jnp.concatenate(hs2, axis=-1)
    out_ref[...] = (jnp.dot(hflat, wfco_ref[...], preferred_element_type=jnp.float32)
                    + jnp.dot(feat_ref[...], wfcf_ref[...],
                              preferred_element_type=jnp.float32)
                    + bfc_ref[...])


def kernel(emb, feat, w1, b1, w2, b2, wih1, whh1, bg1, wihh2, bg2,
           wfco, wfcf, bfc):
    B = emb.shape[0]
    NL = bfc.shape[1]

    NCORES = 2
    Bp = ((B + 8 * NCORES - 1) // (8 * NCORES)) * (8 * NCORES)
    if Bp != B:
        emb = jnp.pad(emb, ((0, Bp - B), (0, 0), (0, 0)))
        feat = jnp.pad(feat, ((0, Bp - B), (0, 0)))
    Bb = Bp // NCORES

    inputs = (emb, feat, w1, b1, w2, b2, wih1, whh1, bg1, wihh2, bg2,
              wfco, wfcf, bfc)

    def bcast_spec(shape):
        nd = len(shape)
        return pl.BlockSpec(shape, lambda i, nd=nd: (0,) * nd)

    in_specs = [
        pl.BlockSpec((Bb,) + emb.shape[1:], lambda i: (i, 0, 0)),
        pl.BlockSpec((Bb,) + feat.shape[1:], lambda i: (i, 0)),
    ] + [bcast_spec(a.shape) for a in inputs[2:]]

    out = pl.pallas_call(
        _cnn_rnn_body,
        out_shape=jax.ShapeDtypeStruct((Bp, NL), jnp.float32),
        grid=(NCORES,),
        in_specs=in_specs,
        out_specs=pl.BlockSpec((Bb, NL), lambda i: (i, 0)),
        compiler_params=pltpu.CompilerParams(
            dimension_semantics=("parallel",)),
    )(*inputs)
    return out[:B]


# speedup vs baseline: 1.0300x; 1.0300x over previous
import jax
import jax.numpy as jnp
from jax.experimental import pallas as pl
from jax.experimental.pallas import tpu as pltpu


def _mm(a3, w):
    B, T, K = a3.shape
    return jnp.dot(a3.reshape(B * T, K), w,
                   preferred_element_type=jnp.float32).reshape(B, T, w.shape[1])


def _cnn_rnn_body(emb_ref, feat_ref,
                  w1_ref, b1_ref,
                  w2_ref, b2_ref,
                  wih1_ref, whh1_ref, bg1_ref,
                  wihh2_ref, bg2_ref,
                  wfco_ref, wfcf_ref, bfc_ref,
                  out_ref):
    bf16 = jnp.bfloat16
    emb = emb_ref[...].astype(bf16)
    B, L, E = emb.shape
    T = L - 2
    C1 = w1_ref.shape[1]
    H = whh1_ref.shape[0]

    win1 = jnp.concatenate(
        [emb[:, 0:T, :], emb[:, 1:T + 1, :], emb[:, 2:T + 2, :]], axis=-1)
    c1 = jax.nn.sigmoid(_mm(win1, w1_ref[...].astype(bf16)) + b1_ref[...])
    c1 = c1.astype(bf16)

    zpad = jnp.zeros((B, 1, C1), bf16)
    c1p = jnp.concatenate([zpad, c1, zpad], axis=1)
    win2 = jnp.concatenate(
        [c1p[:, 0:T, :], c1p[:, 1:T + 1, :], c1p[:, 2:T + 2, :]], axis=-1)
    c2 = jax.nn.sigmoid(_mm(win2, w2_ref[...].astype(bf16)) + b2_ref[...])
    c2 = c2.astype(bf16)

    gates_x1 = _mm(c2, wih1_ref[...].astype(bf16)) + bg1_ref[...]
    whh1 = whh1_ref[...].astype(bf16)
    wih2 = wihh2_ref[0:H, :].astype(bf16)
    whh2 = wihh2_ref[H:2 * H, :].astype(bf16)
    bg2 = bg2_ref[...]

    def apply_gates(gates, c_prev):
        sg = jax.nn.sigmoid(gates)
        tg = jnp.tanh(gates)
        i = sg[:, 0:H]
        f = sg[:, H:2 * H]
        o = sg[:, 3 * H:4 * H]
        g = tg[:, 2 * H:3 * H]
        c_new = f * c_prev + i * g
        h_new = o * jnp.tanh(c_new)
        return h_new, c_new

    h1 = jnp.zeros((B, H), bf16)
    c1s = jnp.zeros((B, H), jnp.float32)
    hs1 = []
    for t in range(T):
        gates1 = gates_x1[:, t, :] + jnp.dot(
            h1, whh1, preferred_element_type=jnp.float32)
        h1f, c1s = apply_gates(gates1, c1s)
        h1 = h1f.astype(bf16)
        hs1.append(h1)

    h1all = jnp.concatenate(hs1, axis=-1).reshape(B, T, H)
    gates_x2 = _mm(h1all, wih2) + bg2

    h2 = jnp.zeros((B, H), bf16)
    c2s = jnp.zeros((B, H), jnp.float32)
    hs2 = []
    for t in range(T):
        gates2 = gates_x2[:, t, :] + jnp.dot(
            h2, whh2, preferred_element_type=jnp.float32)
        h2f, c2s = apply_gates(gates2, c2s)
        h2 = h2f.astype(bf16)
        hs2.append(h2)

    hflat = jnp.concatenate(hs2, axis=-1)
    out_ref[...] = (jnp.dot(hflat, wfco_ref[...].astype(bf16),
                            preferred_element_type=jnp.float32)
                    + jnp.dot(feat_ref[...].astype(bf16),
                              wfcf_ref[...].astype(bf16),
                              preferred_element_type=jnp.float32)
                    + bfc_ref[...])


def kernel(emb, feat, w1, b1, w2, b2, wih1, whh1, bg1, wihh2, bg2,
           wfco, wfcf, bfc):
    B = emb.shape[0]
    NL = bfc.shape[1]

    Bp = max(8, ((B + 7) // 8) * 8)
    if Bp != B:
        emb = jnp.pad(emb, ((0, Bp - B), (0, 0), (0, 0)))
        feat = jnp.pad(feat, ((0, Bp - B), (0, 0)))

    inputs = (emb, feat, w1, b1, w2, b2, wih1, whh1, bg1, wihh2, bg2,
              wfco, wfcf, bfc)

    def full_spec(shape):
        nd = len(shape)
        return pl.BlockSpec(shape, lambda i, nd=nd: (0,) * nd)

    out = pl.pallas_call(
        _cnn_rnn_body,
        out_shape=jax.ShapeDtypeStruct((Bp, NL), jnp.float32),
        grid=(1,),
        in_specs=[full_spec(a.shape) for a in inputs],
        out_specs=full_spec((Bp, NL)),
        compiler_params=pltpu.CompilerParams(
            dimension_semantics=("arbitrary",)),
    )(*inputs)
    return out[:B]


# speedup vs baseline: 1.5417x; 1.4967x over previous
import jax
import jax.numpy as jnp
from jax.experimental import pallas as pl
from jax.experimental.pallas import tpu as pltpu


def _mm(a3, w):
    B, T, K = a3.shape
    return jnp.dot(a3.reshape(B * T, K), w,
                   preferred_element_type=jnp.float32).reshape(B, T, w.shape[1])


def _cnn_rnn_body(emb_ref, feat_ref,
                  w1_ref, b1_ref,
                  w2_ref, b2_ref,
                  wih1_ref, whh1_ref, bg1_ref,
                  wihh2_ref, bg2_ref,
                  wfco_ref, wfcf_ref, bfc_ref,
                  out_ref):
    bf16 = jnp.bfloat16
    f32 = jnp.float32
    emb = emb_ref[...].astype(bf16)
    B, L, E = emb.shape
    T = L - 2
    C1 = w1_ref.shape[1]
    H = whh1_ref.shape[0]

    sv = jnp.concatenate([jnp.full((1, 2 * H), 0.5, f32),
                          jnp.ones((1, H), f32),
                          jnp.full((1, H), 0.5, f32)], axis=1)

    w1f = (w1_ref[...] * 0.5).astype(bf16)
    b1f = b1_ref[...] * 0.5
    w2f = (w2_ref[...] * 0.25).astype(bf16)
    b2f = b2_ref[...] * 0.5 + 0.25 * jnp.sum(w2_ref[...], axis=0,
                                             keepdims=True)
    wih1f = (wih1_ref[...] * (0.5 * sv)).astype(bf16)
    bg1f = sv * (bg1_ref[...] + 0.5 * jnp.sum(wih1_ref[...], axis=0,
                                              keepdims=True))
    whh1f = (whh1_ref[...] * sv).astype(bf16)
    wihh2f = (wihh2_ref[...] * sv).astype(bf16)
    bg2f = bg2_ref[...] * sv

    win1 = jnp.concatenate(
        [emb[:, 0:T, :], emb[:, 1:T + 1, :], emb[:, 2:T + 2, :]], axis=-1)
    t1 = jnp.tanh(_mm(win1, w1f) + b1f).astype(bf16)

    npad = jnp.full((B, 1, C1), -1.0, bf16)
    t1p = jnp.concatenate([npad, t1, npad], axis=1)
    win2 = jnp.concatenate(
        [t1p[:, 0:T, :], t1p[:, 1:T + 1, :], t1p[:, 2:T + 2, :]], axis=-1)
    t2 = jnp.tanh(_mm(win2, w2f) + b2f).astype(bf16)

    u1x = _mm(t2, wih1f) + bg1f

    def apply_gates(tu, c_prev):
        i = 0.5 * tu[:, 0:H] + 0.5
        f = 0.5 * tu[:, H:2 * H] + 0.5
        g = tu[:, 2 * H:3 * H]
        o = 0.5 * tu[:, 3 * H:4 * H] + 0.5
        c_new = f * c_prev + i * g
        h_new = o * jnp.tanh(c_new)
        return h_new, c_new

    h1 = jnp.zeros((B, H), bf16)
    c1s = jnp.zeros((B, H), f32)
    h2 = jnp.zeros((B, H), bf16)
    c2s = jnp.zeros((B, H), f32)

    hs = []
    for t in range(T):
        tu1 = jnp.tanh(u1x[:, t, :] + jnp.dot(
            h1, whh1f, preferred_element_type=f32))
        h1f, c1s = apply_gates(tu1, c1s)
        h1 = h1f.astype(bf16)
        tu2 = jnp.tanh(jnp.dot(jnp.concatenate([h1, h2], axis=-1), wihh2f,
                               preferred_element_type=f32) + bg2f)
        h2f, c2s = apply_gates(tu2, c2s)
        h2 = h2f.astype(bf16)
        hs.append(h2)

    hflat = jnp.concatenate(hs, axis=-1)
    out_ref[...] = (jnp.dot(hflat, wfco_ref[...].astype(bf16),
                            preferred_element_type=f32)
                    + jnp.dot(feat_ref[...].astype(bf16),
                              wfcf_ref[...].astype(bf16),
                              preferred_element_type=f32)
                    + bfc_ref[...])


def kernel(emb, feat, w1, b1, w2, b2, wih1, whh1, bg1, wihh2, bg2,
           wfco, wfcf, bfc):
    B = emb.shape[0]
    NL = bfc.shape[1]

    Bp = max(8, ((B + 7) // 8) * 8)
    if Bp != B:
        emb = jnp.pad(emb, ((0, Bp - B), (0, 0), (0, 0)))
        feat = jnp.pad(feat, ((0, Bp - B), (0, 0)))

    inputs = (emb, feat, w1, b1, w2, b2, wih1, whh1, bg1, wihh2, bg2,
              wfco, wfcf, bfc)

    def full_spec(shape):
        nd = len(shape)
        return pl.BlockSpec(shape, lambda i, nd=nd: (0,) * nd)

    out = pl.pallas_call(
        _cnn_rnn_body,
        out_shape=jax.ShapeDtypeStruct((Bp, NL), jnp.float32),
        grid=(1,),
        in_specs=[full_spec(a.shape) for a in inputs],
        out_specs=full_spec((Bp, NL)),
        compiler_params=pltpu.CompilerParams(
            dimension_semantics=("arbitrary",)),
    )(*inputs)
    return out[:B]
